# baseline (speedup 1.0000x reference)
import functools

import jax
import jax.numpy as jnp
from jax import lax
from jax.experimental import pallas as pl
from jax.experimental.pallas import tpu as pltpu
from jax.experimental.pallas import tpu_sc as plsc

L = 16
NW = 32
D_IN = 66
D_PASS = 64
D_TAB = 128
D_OUT = D_PASS + 2 * D_TAB
SEQ = 50
UNROLL = 5


def _body(emb_hbm, cat_hbm, sub_hbm, out_hbm,
          e0, e1, o0, o1, cat_v, sub_v, ids_s,
          se0, se1, so0, so1, sem_t):
    n_batch = emb_hbm.shape[0]
    per_w = n_batch // NW

    wid = lax.axis_index("s") * 2 + lax.axis_index("c")
    w_base = wid * per_w

    bufs = ((e0, o0, se0, so0), (e1, o1, se1, so1))

    pltpu.make_async_copy(cat_hbm, cat_v, sem_t).start()
    pltpu.make_async_copy(sub_hbm, sub_v, sem_t).start()
    pltpu.make_async_copy(cat_hbm, cat_v, sem_t).wait()
    pltpu.make_async_copy(sub_hbm, sub_v, sem_t).wait()

    def in_copy(g, b):
        return pltpu.make_async_copy(
            emb_hbm.at[w_base + g], bufs[b][0], bufs[b][2])

    def out_copy(g, b):
        return pltpu.make_async_copy(
            bufs[b][1], out_hbm.at[w_base + g], bufs[b][3])

    in_copy(0, 0).start()
    in_copy(1, 1).start()

    def step_pair(h, carry):
        for b in (0, 1):
            g = 2 * h + b
            e, o, se, so = bufs[b]

            in_copy(g, b).wait()

            def extract(r, c):
                v = e[r, pl.ds(D_IN - L, L)].astype(jnp.int32)
                ids_s[2 * r] = v[L - 2]
                ids_s[2 * r + 1] = v[L - 1]
                return c

            lax.fori_loop(0, SEQ, extract, 0)

            @pl.when(g >= 2)
            def _():
                out_copy(g - 2, b).wait()

            def asm(q, c):
                r0 = q * UNROLL
                for dr in range(UNROLL):
                    r = r0 + dr
                    ic = ids_s[2 * r]
                    isub = ids_s[2 * r + 1]
                    for j in range(D_PASS // L):
                        o[r, pl.ds(j * L, L)] = e[r, pl.ds(j * L, L)]
                    for j in range(D_TAB // L):
                        o[r, pl.ds(D_PASS + j * L, L)] = \
                            cat_v[ic, pl.ds(j * L, L)]
                        o[r, pl.ds(D_PASS + D_TAB + j * L, L)] = \
                            sub_v[isub, pl.ds(j * L, L)]
                return c

            lax.fori_loop(0, SEQ // UNROLL, asm, 0)

            @pl.when(g + 2 < per_w)
            def _():
                in_copy(g + 2, b).start()

            out_copy(g, b).start()
        return carry

    lax.fori_loop(0, per_w // 2, step_pair, 0)
    out_copy(per_w - 2, 0).wait()
    out_copy(per_w - 1, 1).wait()


@jax.jit
def kernel(embeddings, cat_table, subcat_table):
    B, S, _ = embeddings.shape

    mesh = plsc.VectorSubcoreMesh(core_axis_name="c", subcore_axis_name="s")
    return pl.kernel(
        _body,
        out_type=jax.ShapeDtypeStruct((B, S, D_OUT), jnp.float32),
        mesh=mesh,
        scratch_types=[
            pltpu.VMEM((SEQ, D_IN), jnp.float32),
            pltpu.VMEM((SEQ, D_IN), jnp.float32),
            pltpu.VMEM((SEQ, D_OUT), jnp.float32),
            pltpu.VMEM((SEQ, D_OUT), jnp.float32),
            pltpu.VMEM((15, D_TAB), jnp.float32),
            pltpu.VMEM((134, D_TAB), jnp.float32),
            pltpu.SMEM((2 * SEQ,), jnp.int32),
            pltpu.SemaphoreType.DMA,
            pltpu.SemaphoreType.DMA,
            pltpu.SemaphoreType.DMA,
            pltpu.SemaphoreType.DMA,
            pltpu.SemaphoreType.DMA,
        ],
    )(embeddings, cat_table, subcat_table)


# speedup vs baseline: 1.0179x; 1.0179x over previous
import functools

import jax
import jax.numpy as jnp
from jax import lax
from jax.experimental import pallas as pl
from jax.experimental.pallas import tpu as pltpu
from jax.experimental.pallas import tpu_sc as plsc

L = 16
NW = 32
D_IN = 66
D_PASS = 64
D_TAB = 128
D_OUT = D_PASS + 2 * D_TAB
SEQ = 50
UNROLL = 5


def _body(emb_hbm, cat_hbm, sub_hbm, out_hbm,
          e0, e1, o0, o1, cat_v, sub_v,
          se0, se1, so0, so1, sem_t):
    n_batch = emb_hbm.shape[0]
    per_w = n_batch // NW

    wid = lax.axis_index("s") * 2 + lax.axis_index("c")
    w_base = wid * per_w

    bufs = ((e0, o0, se0, so0), (e1, o1, se1, so1))

    pltpu.make_async_copy(cat_hbm, cat_v, sem_t).start()
    pltpu.make_async_copy(sub_hbm, sub_v, sem_t).start()
    pltpu.make_async_copy(cat_hbm, cat_v, sem_t).wait()
    pltpu.make_async_copy(sub_hbm, sub_v, sem_t).wait()

    def in_copy(g, b):
        return pltpu.make_async_copy(
            emb_hbm.at[w_base + g], bufs[b][0], bufs[b][2])

    def out_copy(g, b):
        return pltpu.make_async_copy(
            bufs[b][1], out_hbm.at[w_base + g], bufs[b][3])

    in_copy(0, 0).start()
    in_copy(1, 1).start()

    def step_pair(h, carry):
        for b in (0, 1):
            g = 2 * h + b
            e, o, se, so = bufs[b]

            in_copy(g, b).wait()

            @pl.when(g >= 2)
            def _():
                out_copy(g - 2, b).wait()

            def asm(q, c):
                r0 = q * UNROLL
                for dr in range(UNROLL):
                    r = r0 + dr
                    v = e[r, pl.ds(D_IN - L, L)].astype(jnp.int32)
                    ic = v[L - 2]
                    isub = v[L - 1]
                    for j in range(D_PASS // L):
                        o[r, pl.ds(j * L, L)] = e[r, pl.ds(j * L, L)]
                    for j in range(D_TAB // L):
                        o[r, pl.ds(D_PASS + j * L, L)] = \
                            cat_v[ic, pl.ds(j * L, L)]
                        o[r, pl.ds(D_PASS + D_TAB + j * L, L)] = \
                            sub_v[isub, pl.ds(j * L, L)]
                return c

            lax.fori_loop(0, SEQ // UNROLL, asm, 0)

            @pl.when(g + 2 < per_w)
            def _():
                in_copy(g + 2, b).start()

            out_copy(g, b).start()
        return carry

    lax.fori_loop(0, per_w // 2, step_pair, 0)
    out_copy(per_w - 2, 0).wait()
    out_copy(per_w - 1, 1).wait()


@jax.jit
def kernel(embeddings, cat_table, subcat_table):
    B, S, _ = embeddings.shape

    mesh = plsc.VectorSubcoreMesh(core_axis_name="c", subcore_axis_name="s")
    return pl.kernel(
        _body,
        out_type=jax.ShapeDtypeStruct((B, S, D_OUT), jnp.float32),
        mesh=mesh,
        scratch_types=[
            pltpu.VMEM((SEQ, D_IN), jnp.float32),
            pltpu.VMEM((SEQ, D_IN), jnp.float32),
            pltpu.VMEM((SEQ, D_OUT), jnp.float32),
            pltpu.VMEM((SEQ, D_OUT), jnp.float32),
            pltpu.VMEM((15, D_TAB), jnp.float32),
            pltpu.VMEM((134, D_TAB), jnp.float32),
            pltpu.SemaphoreType.DMA,
            pltpu.SemaphoreType.DMA,
            pltpu.SemaphoreType.DMA,
            pltpu.SemaphoreType.DMA,
            pltpu.SemaphoreType.DMA,
        ],
    )(embeddings, cat_table, subcat_table)


# speedup vs baseline: 1.0180x; 1.0002x over previous
import functools

import jax
import jax.numpy as jnp
from jax import lax
from jax.experimental import layout as jex_layout
from jax.experimental import pallas as pl
from jax.experimental.pallas import tpu as pltpu
from jax.experimental.pallas import tpu_sc as plsc

L = 16
NW = 32
D_IN = 66
D_PASS = 64
D_TAB = 128
D_OUT = D_PASS + 2 * D_TAB
SEQ = 50
UNROLL = 5


def _body(emb_hbm, cat_hbm, sub_hbm, out_hbm,
          e0, e1, o0, o1, cat_v, sub_v,
          se0, se1, so0, so1, sem_t):
    n_batch = emb_hbm.shape[0]
    per_w = n_batch // NW

    wid = lax.axis_index("s") * 2 + lax.axis_index("c")
    w_base = wid * per_w

    bufs = ((e0, o0, se0, so0), (e1, o1, se1, so1))

    pltpu.make_async_copy(cat_hbm, cat_v, sem_t).start()
    pltpu.make_async_copy(sub_hbm, sub_v, sem_t).start()
    pltpu.make_async_copy(cat_hbm, cat_v, sem_t).wait()
    pltpu.make_async_copy(sub_hbm, sub_v, sem_t).wait()

    def in_copy(g, b):
        return pltpu.make_async_copy(
            emb_hbm.at[w_base + g], bufs[b][0], bufs[b][2])

    def out_copy(g, b):
        return pltpu.make_async_copy(
            bufs[b][1], out_hbm.at[w_base + g], bufs[b][3])

    in_copy(0, 0).start()
    in_copy(1, 1).start()

    def step_pair(h, carry):
        for b in (0, 1):
            g = 2 * h + b
            e, o, se, so = bufs[b]

            in_copy(g, b).wait()

            @pl.when(g >= 2)
            def _():
                out_copy(g - 2, b).wait()

            def asm(q, c):
                r0 = q * UNROLL
                for dr in range(UNROLL):
                    r = r0 + dr
                    v = e[r, pl.ds(D_IN - L, L)].astype(jnp.int32)
                    ic = v[L - 2]
                    isub = v[L - 1]
                    for j in range(D_PASS // L):
                        o[r, pl.ds(j * L, L)] = e[r, pl.ds(j * L, L)]
                    for j in range(D_TAB // L):
                        o[r, pl.ds(D_PASS + j * L, L)] = \
                            cat_v[ic, pl.ds(j * L, L)]
                        o[r, pl.ds(D_PASS + D_TAB + j * L, L)] = \
                            sub_v[isub, pl.ds(j * L, L)]
                return c

            lax.fori_loop(0, SEQ // UNROLL, asm, 0)

            @pl.when(g + 2 < per_w)
            def _():
                in_copy(g + 2, b).start()

            out_copy(g, b).start()
        return carry

    lax.fori_loop(0, per_w // 2, step_pair, 0)
    out_copy(per_w - 2, 0).wait()
    out_copy(per_w - 1, 1).wait()


def _kernel_impl(embeddings, cat_table, subcat_table):
    B, S, _ = embeddings.shape

    mesh = plsc.VectorSubcoreMesh(core_axis_name="c", subcore_axis_name="s")
    return pl.kernel(
        _body,
        out_type=jax.ShapeDtypeStruct((B, S, D_OUT), jnp.float32),
        mesh=mesh,
        scratch_types=[
            pltpu.VMEM((SEQ, D_IN), jnp.float32),
            pltpu.VMEM((SEQ, D_IN), jnp.float32),
            pltpu.VMEM((SEQ, D_OUT), jnp.float32),
            pltpu.VMEM((SEQ, D_OUT), jnp.float32),
            pltpu.VMEM((15, D_TAB), jnp.float32),
            pltpu.VMEM((134, D_TAB), jnp.float32),
            pltpu.SemaphoreType.DMA,
            pltpu.SemaphoreType.DMA,
            pltpu.SemaphoreType.DMA,
            pltpu.SemaphoreType.DMA,
            pltpu.SemaphoreType.DMA,
        ],
    )(embeddings, cat_table, subcat_table)


@functools.lru_cache(maxsize=None)
def _jitted(sharding):
    fmt = jex_layout.Format(
        jex_layout.Layout(major_to_minor=(0, 1, 2), tiling=()), sharding)
    return jax.jit(_kernel_impl, out_shardings=fmt)


def kernel(embeddings, cat_table, subcat_table):
    try:
        sharding = embeddings.sharding
    except Exception:
        return _kernel_impl(embeddings, cat_table, subcat_table)
    return _jitted(sharding)(embeddings, cat_table, subcat_table)


# speedup vs baseline: 1.0181x; 1.0001x over previous
import functools

import jax
import jax.numpy as jnp
from jax import lax
from jax.experimental import pallas as pl
from jax.experimental.pallas import tpu as pltpu
from jax.experimental.pallas import tpu_sc as plsc

L = 16
NW = 32
D_IN = 66
D_PASS = 64
D_TAB = 128
D_OUT = D_PASS + 2 * D_TAB
SEQ = 50
UNROLL = 5


def _body(emb_hbm, cat_hbm, sub_hbm, out_hbm,
          e0, e1, o0, o1, cat_v, sub_v,
          se0, se1, so0, so1, sem_t):
    n_batch = emb_hbm.shape[0]
    per_w = n_batch // NW

    wid = lax.axis_index("s") * 2 + lax.axis_index("c")
    w_base = wid * per_w

    bufs = ((e0, o0, se0, so0), (e1, o1, se1, so1))

    pltpu.make_async_copy(cat_hbm, cat_v, sem_t).start()
    pltpu.make_async_copy(sub_hbm, sub_v, sem_t).start()
    pltpu.make_async_copy(cat_hbm, cat_v, sem_t).wait()
    pltpu.make_async_copy(sub_hbm, sub_v, sem_t).wait()

    def in_copy(g, b):
        return pltpu.make_async_copy(
            emb_hbm.at[w_base + g], bufs[b][0], bufs[b][2])

    def out_copy(g, b):
        return pltpu.make_async_copy(
            bufs[b][1], out_hbm.at[w_base + g], bufs[b][3])

    in_copy(0, 0).start()
    in_copy(1, 1).start()

    def step_pair(h, carry):
        for b in (0, 1):
            g = 2 * h + b
            e, o, se, so = bufs[b]

            in_copy(g, b).wait()

            @pl.when(g >= 2)
            def _():
                out_copy(g - 2, b).wait()

            def asm(q, c):
                r0 = q * UNROLL
                for dr in range(UNROLL):
                    r = r0 + dr
                    v = e[r, pl.ds(D_IN - L, L)].astype(jnp.int32)
                    ic = v[L - 2]
                    isub = v[L - 1]
                    for j in range(D_PASS // L):
                        o[r, pl.ds(j * L, L)] = e[r, pl.ds(j * L, L)]
                    for j in range(D_TAB // L):
                        o[r, pl.ds(D_PASS + j * L, L)] = \
                            cat_v[ic, pl.ds(j * L, L)]
                        o[r, pl.ds(D_PASS + D_TAB + j * L, L)] = \
                            sub_v[isub, pl.ds(j * L, L)]
                return c

            lax.fori_loop(0, SEQ // UNROLL, asm, 0)

            @pl.when(g + 2 < per_w)
            def _():
                in_copy(g + 2, b).start()

            out_copy(g, b).start()
        return carry

    lax.fori_loop(0, per_w // 2, step_pair, 0)
    out_copy(per_w - 2, 0).wait()
    out_copy(per_w - 1, 1).wait()


def _kernel_impl(embeddings, cat_table, subcat_table):
    B, S, _ = embeddings.shape

    mesh = plsc.VectorSubcoreMesh(core_axis_name="c", subcore_axis_name="s")
    return pl.kernel(
        _body,
        out_type=jax.ShapeDtypeStruct((B, S, D_OUT), jnp.float32),
        mesh=mesh,
        scratch_types=[
            pltpu.VMEM((SEQ, D_IN), jnp.float32),
            pltpu.VMEM((SEQ, D_IN), jnp.float32),
            pltpu.VMEM((SEQ, D_OUT), jnp.float32),
            pltpu.VMEM((SEQ, D_OUT), jnp.float32),
            pltpu.VMEM((15, D_TAB), jnp.float32),
            pltpu.VMEM((134, D_TAB), jnp.float32),
            pltpu.SemaphoreType.DMA,
            pltpu.SemaphoreType.DMA,
            pltpu.SemaphoreType.DMA,
            pltpu.SemaphoreType.DMA,
            pltpu.SemaphoreType.DMA,
        ],
    )(embeddings, cat_table, subcat_table)


kernel = jax.jit(_kernel_impl)
